# baseline (speedup 1.0000x reference)
import jax
import jax.numpy as jnp
from jax import lax
from jax.experimental import pallas as pl
from jax.experimental.pallas import tpu as pltpu
from jax.experimental.pallas import tpu_sc as plsc

N_USERS = 5000
N_ITEMS = 5000
N_EDGES = 320000
D = 128

NC = 2
NS = 16
L = 16

C = 80
EPT = N_EDGES // NS
NCHUNK = EPT // C
NBUF = 5
QD = 3
QG = 2
NOUTER = NCHUNK // NBUF
RC = 200
NRC = N_USERS // RC
ZR = 40
NZC = N_USERS // ZR


def _lane_bcast(wv, e16):
    return lax.gather(
        wv, jnp.full((L, 1), e16, jnp.int32),
        lax.GatherDimensionNumbers(
            offset_dims=(), collapsed_slice_dims=(0,), start_index_map=(0,)),
        slice_sizes=(1,),
        mode=lax.GatherScatterMode.PROMISE_IN_BOUNDS)


def _edge_pass(table_hbm, ew, gsrc, ssrc, acc, gidx, sidx, wring, rows,
               gsem, ssem, msem, sid):
    base = sid * EPT

    def idx_copies(q, qb):
        return (
            pltpu.make_async_copy(gsrc.at[pl.ds(base + q * C, C)],
                                  gidx[qb], msem[qb]),
            pltpu.make_async_copy(ssrc.at[pl.ds(base + q * C, C)],
                                  sidx[qb], msem[qb]),
            pltpu.make_async_copy(ew.at[pl.ds(base + q * C, C)],
                                  wring[qb], msem[qb]),
        )

    def issue_idx(q, qb):
        for d in idx_copies(q, qb):
            d.start()

    def gather_stage(q, qb):
        for d in idx_copies(q, qb):
            d.wait()
        pltpu.async_copy(table_hbm.at[gidx[qb]], rows[qb], gsem[qb])

    def scale(k, b):
        def grp(g, c2):
            wv = wring[b][pl.ds(g * L, L)]
            for e16 in range(L):
                wsp = _lane_bcast(wv, e16)
                e = g * L + e16
                for j in range(D // L):
                    rows[b][e, pl.ds(j * L, L)] = (
                        rows[b][e, pl.ds(j * L, L)] * wsp)
            return c2

        lax.fori_loop(0, C // L, grp, 0)

    def step(k, b):
        pltpu.make_async_copy(table_hbm.at[gidx[b]], rows[b], gsem[b]).wait()
        scale(k, b)
        pltpu.async_copy(rows[b], acc.at[sidx[b]], ssem[b], add=True)

        q2 = k + QD
        qb2 = (b + QD) % NBUF

        @pl.when(q2 < NCHUNK)
        def _():
            @pl.when(k >= NBUF - QD)
            def _():
                pltpu.make_async_copy(rows[qb2], acc.at[sidx[qb2]],
                                      ssem[qb2]).wait()

            issue_idx(q2, qb2)

        q1 = k + QG
        qb1 = (b + QG) % NBUF

        @pl.when(q1 < NCHUNK)
        def _():
            gather_stage(q1, qb1)

    for q in range(QD):
        issue_idx(q, q)
    for q in range(QG):
        gather_stage(q, q)

    def outer(ko, carry):
        for b in range(NBUF):
            step(ko * NBUF + b, b)
        return carry

    lax.fori_loop(0, NOUTER, outer, 0)

    for b in range(NBUF):
        pltpu.make_async_copy(rows[b], acc.at[sidx[b]], ssem[b]).wait()


def _copy_out(acc, out_hbm, sid):
    for k0 in range(2):
        k = sid + 16 * k0

        @pl.when(k < NRC)
        def _():
            pltpu.sync_copy(acc.at[pl.ds(k * RC, RC)],
                            out_hbm.at[pl.ds(k * RC, RC)])


def _sc_body(h_user, h_item, ew, esrc, edst, out_user, out_item, *refs):
    acc = refs[0]
    rows = list(refs[1:1 + NBUF])
    gidx = list(refs[1 + NBUF:1 + 2 * NBUF])
    sidx = list(refs[1 + 2 * NBUF:1 + 3 * NBUF])
    wring = list(refs[1 + 3 * NBUF:1 + 4 * NBUF])
    gsem = list(refs[1 + 4 * NBUF:1 + 5 * NBUF])
    ssem = list(refs[1 + 5 * NBUF:1 + 6 * NBUF])
    msem = list(refs[1 + 6 * NBUF:1 + 7 * NBUF])
    cid = lax.axis_index("c")
    sid = lax.axis_index("s")

    def zero_body(r, carry):
        for j in range(D // L):
            rows[0][r, pl.ds(j * L, L)] = jnp.zeros((L,), jnp.float32)
        return carry

    lax.fori_loop(0, ZR, zero_body, 0)
    for k0 in range(8):
        k = sid + 16 * k0

        @pl.when(k < NZC)
        def _():
            pltpu.sync_copy(rows[0].at[pl.ds(0, ZR)],
                            acc.at[pl.ds(k * ZR, ZR)])

    plsc.subcore_barrier()

    @pl.when(cid == 0)
    def _():
        _edge_pass(h_user, ew, esrc, edst, acc, gidx, sidx, wring, rows,
                   gsem, ssem, msem, sid)

    @pl.when(cid != 0)
    def _():
        _edge_pass(h_item, ew, edst, esrc, acc, gidx, sidx, wring, rows,
                   gsem, ssem, msem, sid)

    plsc.subcore_barrier()

    @pl.when(cid == 0)
    def _():
        _copy_out(acc, out_item, sid)

    @pl.when(cid != 0)
    def _():
        _copy_out(acc, out_user, sid)


@jax.jit
def kernel(h_user, h_item, edge_weight, edge_src, edge_dst):
    mesh = plsc.VectorSubcoreMesh(core_axis_name="c", subcore_axis_name="s",
                                  num_cores=NC, num_subcores=NS)
    f = pl.kernel(
        _sc_body,
        out_type=(
            jax.ShapeDtypeStruct((N_USERS, D), jnp.float32),
            jax.ShapeDtypeStruct((N_ITEMS, D), jnp.float32),
        ),
        mesh=mesh,
        scratch_types=(
            [pltpu.VMEM_SHARED((N_ITEMS, D), jnp.float32)]
            + [pltpu.VMEM((C, D), jnp.float32)] * NBUF
            + [pltpu.VMEM((C,), jnp.int32)] * (2 * NBUF)
            + [pltpu.VMEM((C,), jnp.float32)] * NBUF
            + [pltpu.SemaphoreType.DMA] * (3 * NBUF)
        ),
    )
    return f(h_user, h_item, edge_weight, edge_src, edge_dst)
